# baseline (speedup 1.0000x reference)
import functools

import jax
import jax.numpy as jnp
from jax import lax
from jax.experimental import pallas as pl
from jax.experimental.pallas import tpu as pltpu
from jax.experimental.pallas import tpu_sc as plsc

NC = 2
NS = 16
NW = NC * NS
CHUNK = 16
L = 16


def _make_embed(n_batch: int, n_seq: int, d: int):
  rows_per_w = n_batch // NW
  n_chunks = rows_per_w // CHUNK
  n_bvec = rows_per_w // L
  seq_pad = (n_seq + 7) // 8 * 8
  mesh = plsc.VectorSubcoreMesh(core_axis_name="c", subcore_axis_name="s")

  @functools.partial(
      pl.kernel,
      mesh=mesh,
      out_type=jax.ShapeDtypeStruct((n_batch, n_seq, d), jnp.float32),
      scratch_types=[
          pltpu.VMEM((n_seq, rows_per_w), jnp.int32),
          pltpu.VMEM((rows_per_w * seq_pad,), jnp.int32),
          pltpu.VMEM((2, CHUNK, n_seq, d), jnp.float32),
          pltpu.SemaphoreType.DMA,
          pltpu.SemaphoreType.DMA,
          pltpu.SemaphoreType.DMA,
      ],
      compiler_params=pltpu.CompilerParams(
          use_tc_tiling_on_sc=False, needs_layout_passes=False),
  )
  def embed(table_hbm, tids_hbm, out_hbm, tsl_v, idx_v, chunk_v, sem_g,
            sem_o0, sem_o1):
    wid = lax.axis_index("s") * NC + lax.axis_index("c")
    base = wid * rows_per_w
    pltpu.sync_copy(tids_hbm.at[:, pl.ds(base, rows_per_w)], tsl_v)

    lane_off = lax.iota(jnp.int32, L) * seq_pad

    @pl.loop(0, n_seq)
    def _trans(s):
      for b16 in range(n_bvec):
        vec = tsl_v.at[s][pl.ds(b16 * L, L)]
        plsc.store_scatter(idx_v, [lane_off + (b16 * L * seq_pad + s)], vec)

    def do_chunk(j, buf, sem_o):
      for i in range(CHUNK):
        pltpu.make_async_copy(
            table_hbm.at[idx_v.at[pl.ds((j * CHUNK + i) * seq_pad, n_seq)]],
            chunk_v.at[buf, i], sem_g).start()
      for i in range(CHUNK):
        pltpu.make_async_copy(
            table_hbm.at[idx_v.at[pl.ds((j * CHUNK + i) * seq_pad, n_seq)]],
            chunk_v.at[buf, i], sem_g).wait()
      pltpu.make_async_copy(
          chunk_v.at[buf], out_hbm.at[pl.ds(base + j * CHUNK, CHUNK)],
          sem_o).start()

    def wait_chunk(j, buf, sem_o):
      pltpu.make_async_copy(
          chunk_v.at[buf], out_hbm.at[pl.ds(base + j * CHUNK, CHUNK)],
          sem_o).wait()

    @pl.loop(0, n_chunks, step=2)
    def _step(j0):
      do_chunk(j0, 0, sem_o0)
      do_chunk(j0 + 1, 1, sem_o1)
      wait_chunk(j0, 0, sem_o0)
      wait_chunk(j0 + 1, 1, sem_o1)

  return embed


def kernel(token_ids, W):
  bt, s = token_ids.shape
  n_vocab, d = W.shape
  assert bt % (NW * CHUNK * 2) == 0 and bt % (NW * L) == 0
  tids_t = token_ids.astype(jnp.int32).T
  return _make_embed(bt, s, d)(W, tids_t)


# speedup vs baseline: 1.0244x; 1.0244x over previous
import functools

import jax
import jax.numpy as jnp
from jax import lax
from jax.experimental import pallas as pl
from jax.experimental.pallas import tpu as pltpu
from jax.experimental.pallas import tpu_sc as plsc

NC = 2
NS = 16
NW = NC * NS
BB = 128
K = 8


def _make_embed(n_batch: int, n_seq: int, d: int):
  rows_per_w = n_batch // NW
  n_bb = rows_per_w // BB
  n_steps = n_seq * n_bb
  mesh = plsc.VectorSubcoreMesh(core_axis_name="c", subcore_axis_name="s")

  @functools.partial(
      pl.kernel,
      mesh=mesh,
      out_type=jax.ShapeDtypeStruct((n_batch, n_seq, d), jnp.float32),
      scratch_types=[
          pltpu.VMEM((n_seq, rows_per_w), jnp.int32),
          pltpu.VMEM((K, BB, d), jnp.float32),
          pltpu.SemaphoreType.DMA,
          pltpu.SemaphoreType.DMA,
      ],
      compiler_params=pltpu.CompilerParams(use_tc_tiling_on_sc=False),
  )
  def embed(table_hbm, tids_hbm, out_hbm, tsl_v, rows_v, sem_g, sem_o):
    wid = lax.axis_index("s") * NC + lax.axis_index("c")
    base = wid * rows_per_w
    pltpu.sync_copy(tids_hbm.at[:, pl.ds(base, rows_per_w)], tsl_v)

    def gather(t):
      s = t // n_bb
      bb = t % n_bb
      return pltpu.make_async_copy(
          table_hbm.at[tsl_v.at[s, pl.ds(bb * BB, BB)]],
          rows_v.at[t % K], sem_g)

    def write(t):
      s = t // n_bb
      bb = t % n_bb
      return pltpu.make_async_copy(
          rows_v.at[t % K], out_hbm.at[pl.ds(base + bb * BB, BB), s], sem_o)

    for t in range(K):
      gather(t).start()

    @pl.loop(0, n_steps)
    def _step(t):
      gather(t).wait()
      write(t).start()
      write(t).wait()

      @pl.when(t + K < n_steps)
      def _():
        gather(t + K).start()

  return embed


def kernel(token_ids, W):
  bt, s = token_ids.shape
  n_vocab, d = W.shape
  assert bt % (NW * BB) == 0
  tids_t = token_ids.astype(jnp.int32).T
  return _make_embed(bt, s, d)(W, tids_t)
